# Initial kernel scaffold; baseline (speedup 1.0000x reference)
import jax
import jax.numpy as jnp
from jax.experimental import pallas as pl


def kernel(users, movies, user_table, movie_table, W1, b1, W2, b2, W3, b3, W4, b4, W5, b5, g1, be1, g2, be2, g3, be3, g4, be4):
    raise NotImplementedError("write your pallas kernel here")


# speedup vs baseline: 2.4998x; 2.4998x over previous
import functools
import math

import jax
import jax.numpy as jnp
from jax import lax
from jax.experimental import pallas as pl
from jax.experimental.pallas import tpu as pltpu
from jax.experimental.pallas import tpu_sc as plsc

BATCH = 16384
EMB = 128
_BN_INV = 1.0 / math.sqrt(1.0 + 1e-5)

_NC = 2
_NS = 16
_NW = _NC * _NS
_BPW = BATCH // _NW
_CHUNK = 128
_NCH = _BPW // _CHUNK


def _sc_gather_body(uidx_hbm, midx_hbm, ut_hbm, mt_hbm, ue_out, me_out,
                    idx_v, rows_v, sem):
    wid = lax.axis_index("s") * _NC + lax.axis_index("c")
    base = wid * _BPW
    crow = wid * _NCH
    for idx_hbm, table, out in ((uidx_hbm, ut_hbm, ue_out),
                                (midx_hbm, mt_hbm, me_out)):
        pltpu.sync_copy(idx_hbm.at[pl.ds(crow, _NCH)], idx_v)
        copies = []
        for j in range(_NCH):
            copies.append(pltpu.async_copy(
                table.at[idx_v.at[j]],
                rows_v.at[pl.ds(j * _CHUNK, _CHUNK)], sem))
        for c in copies:
            c.wait()
        pltpu.sync_copy(rows_v, out.at[pl.ds(base, _BPW)])


def _sc_gather(uidx, midx, user_table, movie_table):
    mesh = plsc.VectorSubcoreMesh(core_axis_name="c", subcore_axis_name="s")
    f = pl.kernel(
        _sc_gather_body, mesh=mesh,
        out_type=(jax.ShapeDtypeStruct((BATCH, EMB), jnp.float32),
                  jax.ShapeDtypeStruct((BATCH, EMB), jnp.float32)),
        scratch_types=[
            pltpu.VMEM((_NCH, _CHUNK), jnp.int32),
            pltpu.VMEM((_BPW, EMB), jnp.float32),
            pltpu.SemaphoreType.DMA,
        ],
    )
    return f(uidx, midx, user_table, movie_table)


_BLK = 2048
_NB = BATCH // _BLK


def _mlp_body(ue, me, W1, b1, W2, b2, W3, b3, W4, b4, W5, b5,
              g1, be1, g2, be2, g3, be3, g4, be4, out_ref):
    f32 = jnp.float32
    z = (jnp.dot(ue[...], W1[0:EMB, :], preferred_element_type=f32)
         + jnp.dot(me[...], W1[EMB:2 * EMB, :], preferred_element_type=f32)
         + b1[...])
    h = jnp.maximum(z, 0.0) * (g1[...] * _BN_INV) + be1[...]
    z = jnp.dot(h, W2[...], preferred_element_type=f32) + b2[...]
    h = jnp.maximum(z, 0.0) * (g2[...] * _BN_INV) + be2[...]
    z = jnp.dot(h, W3[...], preferred_element_type=f32) + b3[...]
    h = jnp.maximum(z, 0.0) * (g3[...] * _BN_INV) + be3[...]
    z = jnp.dot(h, W4[...], preferred_element_type=f32) + b4[...]
    h = jnp.maximum(z, 0.0) * (g4[...] * _BN_INV) + be4[...]
    z = jnp.dot(h, W5[...], preferred_element_type=f32) + b5[...]
    out_ref[...] = 1.0 / (1.0 + jnp.exp(-z))


def _mlp(ue, me, W1, b1, W2, b2, W3, b3, W4, b4, W5, b5,
         g1, be1, g2, be2, g3, be3, g4, be4):
    def row_spec(shape):
        return pl.BlockSpec(shape, lambda i: (i, 0))

    def full_spec(shape):
        return pl.BlockSpec(shape, lambda i: (0, 0))

    in_specs = [
        row_spec((_BLK, EMB)), row_spec((_BLK, EMB)),
        full_spec((2 * EMB, 512)), full_spec((1, 512)),
        full_spec((512, 256)), full_spec((1, 256)),
        full_spec((256, 128)), full_spec((1, 128)),
        full_spec((128, 64)), full_spec((1, 64)),
        full_spec((64, 1)), full_spec((1, 1)),
        full_spec((1, 512)), full_spec((1, 512)),
        full_spec((1, 256)), full_spec((1, 256)),
        full_spec((1, 128)), full_spec((1, 128)),
        full_spec((1, 64)), full_spec((1, 64)),
    ]
    return pl.pallas_call(
        _mlp_body,
        grid=(_NB,),
        in_specs=in_specs,
        out_specs=pl.BlockSpec((_BLK, 1), lambda i: (i, 0)),
        out_shape=jax.ShapeDtypeStruct((BATCH, 1), jnp.float32),
    )(ue, me, W1, b1.reshape(1, -1), W2, b2.reshape(1, -1),
      W3, b3.reshape(1, -1), W4, b4.reshape(1, -1), W5, b5.reshape(1, -1),
      g1.reshape(1, -1), be1.reshape(1, -1), g2.reshape(1, -1),
      be2.reshape(1, -1), g3.reshape(1, -1), be3.reshape(1, -1),
      g4.reshape(1, -1), be4.reshape(1, -1))


def kernel(users, movies, user_table, movie_table, W1, b1, W2, b2, W3, b3,
           W4, b4, W5, b5, g1, be1, g2, be2, g3, be3, g4, be4):
    uidx = users.astype(jnp.int32).reshape(BATCH // _CHUNK, _CHUNK)
    midx = movies.astype(jnp.int32).reshape(BATCH // _CHUNK, _CHUNK)
    ue, me = _sc_gather(uidx, midx, user_table, movie_table)
    out = _mlp(ue, me, W1, b1, W2, b2, W3, b3, W4, b4, W5, b5,
               g1, be1, g2, be2, g3, be3, g4, be4)
    return out.reshape(BATCH)


# speedup vs baseline: 2.5355x; 1.0143x over previous
import functools
import math

import jax
import jax.numpy as jnp
from jax import lax
from jax.experimental import pallas as pl
from jax.experimental.pallas import tpu as pltpu
from jax.experimental.pallas import tpu_sc as plsc

BATCH = 16384
EMB = 128
_BN_INV = 1.0 / math.sqrt(1.0 + 1e-5)

_NC = 2
_NS = 16
_NW = _NC * _NS
_NSPLIT = 2
_SUB = BATCH // _NSPLIT
_BPW = _SUB // _NW
_CHUNK = 128
_NCH = _BPW // _CHUNK


def _sc_gather_body(uidx_hbm, midx_hbm, ut_hbm, mt_hbm, ue_out, me_out,
                    idx_v, rows_v, sem):
    wid = lax.axis_index("s") * _NC + lax.axis_index("c")
    base = wid * _BPW
    crow = wid * _NCH
    for idx_hbm, table, out in ((uidx_hbm, ut_hbm, ue_out),
                                (midx_hbm, mt_hbm, me_out)):
        pltpu.sync_copy(idx_hbm.at[pl.ds(crow, _NCH)], idx_v)
        copies = []
        for j in range(_NCH):
            copies.append(pltpu.async_copy(
                table.at[idx_v.at[j]],
                rows_v.at[pl.ds(j * _CHUNK, _CHUNK)], sem))
        for c in copies:
            c.wait()
        pltpu.sync_copy(rows_v, out.at[pl.ds(base, _BPW)])


def _sc_gather(uidx, midx, user_table, movie_table):
    mesh = plsc.VectorSubcoreMesh(core_axis_name="c", subcore_axis_name="s")
    f = pl.kernel(
        _sc_gather_body, mesh=mesh,
        out_type=(jax.ShapeDtypeStruct((_SUB, EMB), jnp.float32),
                  jax.ShapeDtypeStruct((_SUB, EMB), jnp.float32)),
        scratch_types=[
            pltpu.VMEM((_NCH, _CHUNK), jnp.int32),
            pltpu.VMEM((_BPW, EMB), jnp.float32),
            pltpu.SemaphoreType.DMA,
        ],
    )
    return f(uidx, midx, user_table, movie_table)


_BLK = 2048
_NB = _SUB // _BLK


def _mlp_body(ue, me, W1, b1, W2, b2, W3, b3, W4, b4, W5, b5,
              g1, be1, g2, be2, g3, be3, g4, be4, out_ref):
    f32 = jnp.float32
    z = (jnp.dot(ue[...], W1[0:EMB, :], preferred_element_type=f32)
         + jnp.dot(me[...], W1[EMB:2 * EMB, :], preferred_element_type=f32)
         + b1[...])
    h = jnp.maximum(z, 0.0) * (g1[...] * _BN_INV) + be1[...]
    z = jnp.dot(h, W2[...], preferred_element_type=f32) + b2[...]
    h = jnp.maximum(z, 0.0) * (g2[...] * _BN_INV) + be2[...]
    z = jnp.dot(h, W3[...], preferred_element_type=f32) + b3[...]
    h = jnp.maximum(z, 0.0) * (g3[...] * _BN_INV) + be3[...]
    z = jnp.dot(h, W4[...], preferred_element_type=f32) + b4[...]
    h = jnp.maximum(z, 0.0) * (g4[...] * _BN_INV) + be4[...]
    z = jnp.dot(h, W5[...], preferred_element_type=f32) + b5[...]
    out_ref[...] = 1.0 / (1.0 + jnp.exp(-z))


def _mlp(ue, me, W1, b1, W2, b2, W3, b3, W4, b4, W5, b5,
         g1, be1, g2, be2, g3, be3, g4, be4):
    def row_spec(shape):
        return pl.BlockSpec(shape, lambda i: (i, 0))

    def full_spec(shape):
        return pl.BlockSpec(shape, lambda i: (0, 0))

    in_specs = [
        row_spec((_BLK, EMB)), row_spec((_BLK, EMB)),
        full_spec((2 * EMB, 512)), full_spec((1, 512)),
        full_spec((512, 256)), full_spec((1, 256)),
        full_spec((256, 128)), full_spec((1, 128)),
        full_spec((128, 64)), full_spec((1, 64)),
        full_spec((64, 1)), full_spec((1, 1)),
        full_spec((1, 512)), full_spec((1, 512)),
        full_spec((1, 256)), full_spec((1, 256)),
        full_spec((1, 128)), full_spec((1, 128)),
        full_spec((1, 64)), full_spec((1, 64)),
    ]
    return pl.pallas_call(
        _mlp_body,
        grid=(_NB,),
        in_specs=in_specs,
        out_specs=pl.BlockSpec((_BLK, 1), lambda i: (i, 0)),
        out_shape=jax.ShapeDtypeStruct((_SUB, 1), jnp.float32),
    )(ue, me, W1, b1.reshape(1, -1), W2, b2.reshape(1, -1),
      W3, b3.reshape(1, -1), W4, b4.reshape(1, -1), W5, b5.reshape(1, -1),
      g1.reshape(1, -1), be1.reshape(1, -1), g2.reshape(1, -1),
      be2.reshape(1, -1), g3.reshape(1, -1), be3.reshape(1, -1),
      g4.reshape(1, -1), be4.reshape(1, -1))


def kernel(users, movies, user_table, movie_table, W1, b1, W2, b2, W3, b3,
           W4, b4, W5, b5, g1, be1, g2, be2, g3, be3, g4, be4):
    uidx = users.astype(jnp.int32).reshape(_NSPLIT, _SUB // _CHUNK, _CHUNK)
    midx = movies.astype(jnp.int32).reshape(_NSPLIT, _SUB // _CHUNK, _CHUNK)
    outs = []
    for c in range(_NSPLIT):
        ue, me = _sc_gather(uidx[c], midx[c], user_table, movie_table)
        outs.append(_mlp(ue, me, W1, b1, W2, b2, W3, b3, W4, b4, W5, b5,
                         g1, be1, g2, be2, g3, be3, g4, be4))
    return jnp.concatenate(outs, axis=0).reshape(BATCH)
